# Initial kernel scaffold; baseline (speedup 1.0000x reference)
import jax
import jax.numpy as jnp
from jax.experimental import pallas as pl


def kernel(node_feats, edge_feats, edge_index, W_node, b_node, W_e1, b_e1, W_e2, b_e2, W_out, b_out):
    raise NotImplementedError("write your pallas kernel here")


# speedup vs baseline: 2.3027x; 2.3027x over previous
import functools
import math

import jax
import jax.numpy as jnp
from jax import lax
from jax.experimental import pallas as pl
from jax.experimental.pallas import tpu as pltpu
from jax.experimental.pallas import tpu_sc as plsc

N = 10000
E = 320000
D_NODE = 128
D_EDGE = 16
HID = 64
OUT = 128
_LOG2 = math.log(2.0)


def _ssp(x):
    return jnp.maximum(x, 0.0) + jnp.log1p(jnp.exp(-jnp.abs(x))) - _LOG2


def _hv_body(x_ref, w_ref, b_ref, o_ref):
    hv = (
        jnp.dot(x_ref[...], w_ref[...], preferred_element_type=jnp.float32)
        + b_ref[...]
    )
    o_ref[...] = jnp.concatenate([hv, jnp.zeros_like(hv)], axis=1)


def _he_body(ef_ref, w1_ref, b1_ref, w2_ref, b2_ref, o_ref):
    h = jnp.dot(ef_ref[...], w1_ref[...], preferred_element_type=jnp.float32)
    h = _ssp(h + b1_ref[...])
    h = jnp.dot(h, w2_ref[...], preferred_element_type=jnp.float32)
    o_ref[...] = _ssp(h + b2_ref[...])


def _out_body(p_ref, w_ref, b_ref, o_ref):
    agg = p_ref[0, :, :HID] + p_ref[1, :, :HID]
    h = jnp.dot(agg, w_ref[...], preferred_element_type=jnp.float32)
    o_ref[...] = _ssp(h + b_ref[...])


_hv_call = pl.pallas_call(
    _hv_body,
    out_shape=jax.ShapeDtypeStruct((N, 2 * HID), jnp.float32),
)

_E_BLK = 8000
_he_call = pl.pallas_call(
    _he_body,
    grid=(E // _E_BLK,),
    in_specs=[
        pl.BlockSpec((_E_BLK, D_EDGE), lambda i: (i, 0)),
        pl.BlockSpec((D_EDGE, HID), lambda i: (0, 0)),
        pl.BlockSpec((1, HID), lambda i: (0, 0)),
        pl.BlockSpec((HID, HID), lambda i: (0, 0)),
        pl.BlockSpec((1, HID), lambda i: (0, 0)),
    ],
    out_specs=pl.BlockSpec((_E_BLK, HID), lambda i: (i, 0)),
    out_shape=jax.ShapeDtypeStruct((E, HID), jnp.float32),
)

_out_call = pl.pallas_call(
    _out_body,
    out_shape=jax.ShapeDtypeStruct((N, OUT), jnp.float32),
)


_info = plsc.get_sparse_core_info()
_NC = _info.num_cores
_NS = _info.num_subcores
_L = _info.num_lanes
_NW = _NC * _NS
_EPW = E // _NW
_CH = 80
_NCH = _EPW // _CH
assert _NCH * _CH == _EPW
_NBLK = N // _CH
_BPS = -(-_NBLK // _NS)


def _sc_body(hv_hbm, src_hbm, dst_hbm, he_hbm, out_hbm,
             src_v, dst_v, hv_v, he_v, m_v, sem, agg_sh):
    cid = lax.axis_index("c")
    sid = lax.axis_index("s")
    wid = sid * _NC + cid

    zvec = jnp.zeros((_L,), jnp.float32)

    def zrow(r, _c):
        for k in range(2 * HID // _L):
            m_v[r, pl.ds(k * _L, _L)] = zvec
        return _c

    lax.fori_loop(0, _CH, zrow, 0)

    def init_blk(j, _c):
        b = lax.rem(sid + _NS * j, _NBLK)
        pltpu.sync_copy(m_v, agg_sh.at[pl.ds(b * _CH, _CH)])
        return _c

    lax.fori_loop(0, _BPS, init_blk, 0)
    plsc.subcore_barrier()

    base = wid * _EPW

    def chunk(i, _):
        ebase = base + i * _CH
        pltpu.sync_copy(src_hbm.at[pl.ds(ebase, _CH)], src_v)
        pltpu.sync_copy(dst_hbm.at[pl.ds(ebase, _CH)], dst_v)
        pltpu.async_copy(hv_hbm.at[src_v], hv_v, sem).wait()
        pltpu.sync_copy(he_hbm.at[pl.ds(ebase, _CH)], he_v)

        def row(r, _c):
            for k in range(HID // _L):
                s = pl.ds(k * _L, _L)
                m_v[r, s] = he_v[r, s] * hv_v[r, s]
            return _c

        lax.fori_loop(0, _CH, row, 0)
        pltpu.sync_copy(m_v, agg_sh.at[dst_v], add=True)
        return _

    lax.fori_loop(0, _NCH, chunk, 0)
    plsc.subcore_barrier()

    def out_blk(j, _c):
        b = lax.rem(sid + _NS * j, _NBLK)
        r0 = b * _CH
        pltpu.sync_copy(agg_sh.at[pl.ds(r0, _CH)], m_v)
        pltpu.sync_copy(m_v, out_hbm.at[cid, pl.ds(r0, _CH)])
        return _c

    lax.fori_loop(0, _BPS, out_blk, 0)


_sc_call = pl.kernel(
    _sc_body,
    out_type=jax.ShapeDtypeStruct((_NC, N, 2 * HID), jnp.float32),
    mesh=plsc.VectorSubcoreMesh(core_axis_name="c", subcore_axis_name="s"),
    scratch_types=[
        pltpu.VMEM((_CH,), jnp.int32),
        pltpu.VMEM((_CH,), jnp.int32),
        pltpu.VMEM((_CH, 2 * HID), jnp.float32),
        pltpu.VMEM((_CH, HID), jnp.float32),
        pltpu.VMEM((_CH, 2 * HID), jnp.float32),
        pltpu.SemaphoreType.DMA,
        pltpu.VMEM_SHARED((N, 2 * HID), jnp.float32),
    ],
)


def kernel(node_feats, edge_feats, edge_index, W_node, b_node,
           W_e1, b_e1, W_e2, b_e2, W_out, b_out):
    src = edge_index[0]
    dst = edge_index[1]
    hv = _hv_call(node_feats, W_node, b_node.reshape(1, HID))
    he = _he_call(edge_feats, W_e1, b_e1.reshape(1, HID),
                  W_e2, b_e2.reshape(1, HID))
    partials = _sc_call(hv, src, dst, he)
    h_out = _out_call(partials, W_out, b_out.reshape(1, OUT))
    return (h_out, he)


# speedup vs baseline: 2.6060x; 1.1317x over previous
import functools
import math

import jax
import jax.numpy as jnp
from jax import lax
from jax.experimental import pallas as pl
from jax.experimental.pallas import tpu as pltpu
from jax.experimental.pallas import tpu_sc as plsc

N = 10000
E = 320000
D_NODE = 128
D_EDGE = 16
HID = 64
OUT = 128
_LOG2 = math.log(2.0)


def _ssp(x):
    return jnp.maximum(x, 0.0) + jnp.log1p(jnp.exp(-jnp.abs(x))) - _LOG2


def _hv_body(x_ref, w_ref, b_ref, o_ref):
    hv = (
        jnp.dot(x_ref[...], w_ref[...], preferred_element_type=jnp.float32)
        + b_ref[...]
    )
    o_ref[...] = jnp.concatenate([hv, jnp.zeros_like(hv)], axis=1)


def _he_body(ef_ref, w1_ref, b1_ref, w2_ref, b2_ref, o_ref):
    h = jnp.dot(ef_ref[...], w1_ref[...], preferred_element_type=jnp.float32)
    h = _ssp(h + b1_ref[...])
    h = jnp.dot(h, w2_ref[...], preferred_element_type=jnp.float32)
    o_ref[...] = _ssp(h + b2_ref[...])


def _out_body(p_ref, w_ref, b_ref, o_ref):
    agg = p_ref[0, :, :HID] + p_ref[1, :, :HID]
    h = jnp.dot(agg, w_ref[...], preferred_element_type=jnp.float32)
    o_ref[...] = _ssp(h + b_ref[...])


_hv_call = pl.pallas_call(
    _hv_body,
    out_shape=jax.ShapeDtypeStruct((N, 2 * HID), jnp.float32),
)

_E_BLK = 8000
_he_call = pl.pallas_call(
    _he_body,
    grid=(E // _E_BLK,),
    in_specs=[
        pl.BlockSpec((_E_BLK, D_EDGE), lambda i: (i, 0)),
        pl.BlockSpec((D_EDGE, HID), lambda i: (0, 0)),
        pl.BlockSpec((1, HID), lambda i: (0, 0)),
        pl.BlockSpec((HID, HID), lambda i: (0, 0)),
        pl.BlockSpec((1, HID), lambda i: (0, 0)),
    ],
    out_specs=pl.BlockSpec((_E_BLK, HID), lambda i: (i, 0)),
    out_shape=jax.ShapeDtypeStruct((E, HID), jnp.float32),
)

_out_call = pl.pallas_call(
    _out_body,
    out_shape=jax.ShapeDtypeStruct((N, OUT), jnp.float32),
)


_info = plsc.get_sparse_core_info()
_NC = _info.num_cores
_NS = _info.num_subcores
_L = _info.num_lanes
_NW = _NC * _NS
_EPW = E // _NW
_CH = 80
_NCH = _EPW // _CH
assert _NCH * _CH == _EPW
_NBLK = N // _CH
_BPS = -(-_NBLK // _NS)


def _sc_body(hv_hbm, src_hbm, dst_hbm, he_hbm, out_hbm,
             src_v0, dst_v0, hv_v0, src_v1, dst_v1, hv_v1,
             he_v, m_v, gsem0, gsem1, agg_sh):
    bufs = ((src_v0, dst_v0, hv_v0, gsem0),
            (src_v1, dst_v1, hv_v1, gsem1))
    cid = lax.axis_index("c")
    sid = lax.axis_index("s")
    wid = sid * _NC + cid

    zvec = jnp.zeros((_L,), jnp.float32)

    def zrow(r, _c):
        for k in range(2 * HID // _L):
            m_v[r, pl.ds(k * _L, _L)] = zvec
        return _c

    lax.fori_loop(0, _CH, zrow, 0)

    def init_blk(j, _c):
        b = lax.rem(sid + _NS * j, _NBLK)
        pltpu.sync_copy(m_v, agg_sh.at[pl.ds(b * _CH, _CH)])
        return _c

    lax.fori_loop(0, _BPS, init_blk, 0)
    plsc.subcore_barrier()

    base = wid * _EPW

    def load(i, b):
        src_v, dst_v, hv_v, gsem = bufs[b]
        ebase = base + i * _CH
        pltpu.sync_copy(src_hbm.at[pl.ds(ebase, _CH)], src_v)
        pltpu.sync_copy(dst_hbm.at[pl.ds(ebase, _CH)], dst_v)
        pltpu.async_copy(hv_hbm.at[src_v], hv_v, gsem)

    def compute(i, b):
        src_v, dst_v, hv_v, gsem = bufs[b]
        pltpu.sync_copy(he_hbm.at[pl.ds(base + i * _CH, _CH)], he_v)
        pltpu.make_async_copy(hv_hbm.at[src_v], hv_v, gsem).wait()

        def row(r, _c):
            for k in range(HID // _L):
                s = pl.ds(k * _L, _L)
                m_v[r, s] = he_v[r, s] * hv_v[r, s]
            return _c

        lax.fori_loop(0, _CH, row, 0)
        pltpu.sync_copy(m_v, agg_sh.at[dst_v], add=True)

    load(0, 0)

    def pair(g, _c):
        i0 = 2 * g
        load(i0 + 1, 1)
        compute(i0, 0)
        load(i0 + 2, 0)
        compute(i0 + 1, 1)
        return _c

    lax.fori_loop(0, (_NCH - 1) // 2, pair, 0)
    compute(_NCH - 1, 0)
    plsc.subcore_barrier()

    def out_blk(j, _c):
        b = lax.rem(sid + _NS * j, _NBLK)
        r0 = b * _CH
        pltpu.sync_copy(agg_sh.at[pl.ds(r0, _CH)], m_v)
        pltpu.sync_copy(m_v, out_hbm.at[cid, pl.ds(r0, _CH)])
        return _c

    lax.fori_loop(0, _BPS, out_blk, 0)


_sc_call = pl.kernel(
    _sc_body,
    out_type=jax.ShapeDtypeStruct((_NC, N, 2 * HID), jnp.float32),
    mesh=plsc.VectorSubcoreMesh(core_axis_name="c", subcore_axis_name="s"),
    scratch_types=[
        pltpu.VMEM((_CH,), jnp.int32),
        pltpu.VMEM((_CH,), jnp.int32),
        pltpu.VMEM((_CH, 2 * HID), jnp.float32),
        pltpu.VMEM((_CH,), jnp.int32),
        pltpu.VMEM((_CH,), jnp.int32),
        pltpu.VMEM((_CH, 2 * HID), jnp.float32),
        pltpu.VMEM((_CH, HID), jnp.float32),
        pltpu.VMEM((_CH, 2 * HID), jnp.float32),
        pltpu.SemaphoreType.DMA,
        pltpu.SemaphoreType.DMA,
        pltpu.VMEM_SHARED((N, 2 * HID), jnp.float32),
    ],
)


def kernel(node_feats, edge_feats, edge_index, W_node, b_node,
           W_e1, b_e1, W_e2, b_e2, W_out, b_out):
    src = edge_index[0]
    dst = edge_index[1]
    hv = _hv_call(node_feats, W_node, b_node.reshape(1, HID))
    he = _he_call(edge_feats, W_e1, b_e1.reshape(1, HID),
                  W_e2, b_e2.reshape(1, HID))
    partials = _sc_call(hv, src, dst, he)
    h_out = _out_call(partials, W_out, b_out.reshape(1, OUT))
    return (h_out, he)


# speedup vs baseline: 2.6102x; 1.0016x over previous
import functools
import math

import jax
import jax.numpy as jnp
from jax import lax
from jax.experimental import pallas as pl
from jax.experimental.pallas import tpu as pltpu
from jax.experimental.pallas import tpu_sc as plsc

N = 10000
E = 320000
D_NODE = 128
D_EDGE = 16
HID = 64
OUT = 128
_LOG2 = math.log(2.0)


def _ssp(x):
    return jnp.maximum(x, 0.0) + jnp.log1p(jnp.exp(-jnp.abs(x))) - _LOG2


def _hv_body(x_ref, w_ref, b_ref, o_ref):
    hv = (
        jnp.dot(x_ref[...], w_ref[...], preferred_element_type=jnp.float32)
        + b_ref[...]
    )
    o_ref[...] = jnp.concatenate([hv, jnp.zeros_like(hv)], axis=1)


def _he_body(ef_ref, w1_ref, b1_ref, w2_ref, b2_ref, o_ref):
    h = jnp.dot(ef_ref[...], w1_ref[...], preferred_element_type=jnp.float32)
    h = _ssp(h + b1_ref[...])
    h = jnp.dot(h, w2_ref[...], preferred_element_type=jnp.float32)
    o_ref[...] = _ssp(h + b2_ref[...])


def _out_body(p_ref, w_ref, b_ref, o_ref):
    agg = p_ref[0, :, :HID] + p_ref[1, :, :HID]
    h = jnp.dot(agg, w_ref[...], preferred_element_type=jnp.float32)
    o_ref[...] = _ssp(h + b_ref[...])


_hv_call = pl.pallas_call(
    _hv_body,
    out_shape=jax.ShapeDtypeStruct((N, 2 * HID), jnp.float32),
)

_E_BLK = 8000
_he_call = pl.pallas_call(
    _he_body,
    grid=(E // _E_BLK,),
    in_specs=[
        pl.BlockSpec((_E_BLK, D_EDGE), lambda i: (i, 0)),
        pl.BlockSpec((D_EDGE, HID), lambda i: (0, 0)),
        pl.BlockSpec((1, HID), lambda i: (0, 0)),
        pl.BlockSpec((HID, HID), lambda i: (0, 0)),
        pl.BlockSpec((1, HID), lambda i: (0, 0)),
    ],
    out_specs=pl.BlockSpec((_E_BLK, HID), lambda i: (i, 0)),
    out_shape=jax.ShapeDtypeStruct((E, HID), jnp.float32),
)

_out_call = pl.pallas_call(
    _out_body,
    out_shape=jax.ShapeDtypeStruct((N, OUT), jnp.float32),
)


_info = plsc.get_sparse_core_info()
_NC = _info.num_cores
_NS = _info.num_subcores
_L = _info.num_lanes
_NW = _NC * _NS
_EPW = E // _NW
_CH = 80
_NCH = _EPW // _CH
assert _NCH * _CH == _EPW
_NBLK = N // _CH
_BPS = -(-_NBLK // _NS)


def _sc_body(hv_hbm, src_hbm, dst_hbm, he_hbm, out_hbm,
             src_v0, dst_v0, hv_v0, src_v1, dst_v1, hv_v1,
             he_v, m_v, gsem0, gsem1, agg_sh):
    bufs = ((src_v0, dst_v0, hv_v0, gsem0),
            (src_v1, dst_v1, hv_v1, gsem1))
    cid = lax.axis_index("c")
    sid = lax.axis_index("s")
    wid = sid * _NC + cid

    zvec = jnp.zeros((_L,), jnp.float32)

    def zrow(r, _c):
        for k in range(2 * HID // _L):
            m_v[r, pl.ds(k * _L, _L)] = zvec
        return _c

    lax.fori_loop(0, _CH, zrow, 0)

    def init_blk(j, _c):
        b = lax.rem(sid + _NS * j, _NBLK)
        pltpu.sync_copy(m_v, agg_sh.at[pl.ds(b * _CH, _CH)])
        return _c

    lax.fori_loop(0, _BPS, init_blk, 0)
    plsc.subcore_barrier()

    base = wid * _EPW

    def load(i, b):
        src_v, dst_v, hv_v, gsem = bufs[b]
        ebase = base + i * _CH
        pltpu.sync_copy(src_hbm.at[pl.ds(ebase, _CH)], src_v)
        pltpu.sync_copy(dst_hbm.at[pl.ds(ebase, _CH)], dst_v)
        pltpu.async_copy(hv_hbm.at[src_v], hv_v, gsem)

    def compute(i, b):
        src_v, dst_v, hv_v, gsem = bufs[b]
        pltpu.sync_copy(he_hbm.at[pl.ds(base + i * _CH, _CH)], he_v)
        pltpu.make_async_copy(hv_hbm.at[src_v], hv_v, gsem).wait()

        def row(r4, _c):
            for u in range(4):
                r = 4 * r4 + u
                for k in range(HID // _L):
                    s = pl.ds(k * _L, _L)
                    m_v[r, s] = he_v[r, s] * hv_v[r, s]
            return _c

        lax.fori_loop(0, _CH // 4, row, 0)
        pltpu.sync_copy(m_v, agg_sh.at[dst_v], add=True)

    load(0, 0)

    def pair(g, _c):
        i0 = 2 * g
        load(i0 + 1, 1)
        compute(i0, 0)
        load(i0 + 2, 0)
        compute(i0 + 1, 1)
        return _c

    lax.fori_loop(0, (_NCH - 1) // 2, pair, 0)
    compute(_NCH - 1, 0)
    plsc.subcore_barrier()

    def out_blk(j, _c):
        b = lax.rem(sid + _NS * j, _NBLK)
        r0 = b * _CH
        pltpu.sync_copy(agg_sh.at[pl.ds(r0, _CH)], m_v)
        pltpu.sync_copy(m_v, out_hbm.at[cid, pl.ds(r0, _CH)])
        return _c

    lax.fori_loop(0, _BPS, out_blk, 0)


_sc_call = pl.kernel(
    _sc_body,
    out_type=jax.ShapeDtypeStruct((_NC, N, 2 * HID), jnp.float32),
    mesh=plsc.VectorSubcoreMesh(core_axis_name="c", subcore_axis_name="s"),
    scratch_types=[
        pltpu.VMEM((_CH,), jnp.int32),
        pltpu.VMEM((_CH,), jnp.int32),
        pltpu.VMEM((_CH, 2 * HID), jnp.float32),
        pltpu.VMEM((_CH,), jnp.int32),
        pltpu.VMEM((_CH,), jnp.int32),
        pltpu.VMEM((_CH, 2 * HID), jnp.float32),
        pltpu.VMEM((_CH, HID), jnp.float32),
        pltpu.VMEM((_CH, 2 * HID), jnp.float32),
        pltpu.SemaphoreType.DMA,
        pltpu.SemaphoreType.DMA,
        pltpu.VMEM_SHARED((N, 2 * HID), jnp.float32),
    ],
)


def kernel(node_feats, edge_feats, edge_index, W_node, b_node,
           W_e1, b_e1, W_e2, b_e2, W_out, b_out):
    src = edge_index[0]
    dst = edge_index[1]
    hv = _hv_call(node_feats, W_node, b_node.reshape(1, HID))
    he = _he_call(edge_feats, W_e1, b_e1.reshape(1, HID),
                  W_e2, b_e2.reshape(1, HID))
    partials = _sc_call(hv, src, dst, he)
    h_out = _out_call(partials, W_out, b_out.reshape(1, OUT))
    return (h_out, he)
